# baseline (speedup 1.0000x reference)
import jax
import jax.numpy as jnp
from jax.experimental import pallas as pl
from jax.experimental.pallas import tpu as pltpu
from jax.experimental.pallas import tpu_sc as plsc

FF = 2048
E = 8
TOPK = 2
BLK = 512
BM_ROUTER = 1024
SC_W = 128


def _router_body(x_ref, wr_ref, logits_ref):
    x = x_ref[...].astype(jnp.bfloat16)
    w = wr_ref[...].astype(jnp.bfloat16)
    logits_ref[...] = jax.lax.dot_general(
        x, w, (((1,), (1,)), ((), ())), preferred_element_type=jnp.float32)


def _moe_body(be_ref, x_ref, win_ref, wout_ref, out_ref):
    x = x_ref[...].astype(jnp.bfloat16)
    win = win_ref[0]
    h = jax.lax.dot_general(
        x, win, (((1,), (1,)), ((), ())), preferred_element_type=jnp.float32)
    g = h[:, :FF]
    u = h[:, FF:]
    a = (g * jax.nn.sigmoid(g) * u).astype(jnp.bfloat16)
    wout = wout_ref[0]
    out_ref[...] = jax.lax.dot_general(
        a, wout, (((1,), (1,)), ((), ())),
        preferred_element_type=jnp.float32).astype(jnp.bfloat16)


def _sc_dispatch(x_sub, pos_even, pos_odd, P):
    n_sub, dsub = x_sub.shape
    mesh = plsc.VectorSubcoreMesh(core_axis_name="core", subcore_axis_name="subcore")

    @pl.kernel(out_type=jax.ShapeDtypeStruct((P * 8, dsub), x_sub.dtype), mesh=mesh)
    def dispatch_kernel(x_hbm, ie_hbm, io_hbm, o_hbm):
        def body(x_vmem, ie_vmem, io_vmem):
            pltpu.sync_copy(x_vmem, o_hbm.at[ie_vmem.at[0]])
            pltpu.sync_copy(x_vmem, o_hbm.at[io_vmem.at[0]])

        pltpu.emit_pipeline(
            body,
            grid=(n_sub // SC_W,),
            in_specs=[
                pl.BlockSpec((SC_W, dsub), lambda i: (i, 0)),
                pl.BlockSpec((1, SC_W), lambda i: (0, i)),
                pl.BlockSpec((1, SC_W), lambda i: (0, i)),
            ],
            out_specs=[],
            core_axis_name=("core", "subcore"),
            dimension_semantics=(pltpu.PARALLEL,),
        )(x_hbm, ie_hbm, io_hbm)

    return dispatch_kernel(x_sub, pos_even.reshape(1, n_sub), pos_odd.reshape(1, n_sub))


def kernel(layer_input, w_router, w_in, w_out):
    bsz, length, d = layer_input.shape
    T = bsz * length
    S = T * TOPK
    P = S + E * BLK
    NB = P // BLK
    x = layer_input.reshape(T, d)

    logits = pl.pallas_call(
        _router_body,
        grid=(T // BM_ROUTER,),
        in_specs=[
            pl.BlockSpec((BM_ROUTER, d), lambda i: (i, 0)),
            pl.BlockSpec((E, d), lambda i: (0, 0)),
        ],
        out_specs=pl.BlockSpec((BM_ROUTER, E), lambda i: (i, 0)),
        out_shape=jax.ShapeDtypeStruct((T, E), jnp.float32),
    )(x, w_router)

    top_vals, top_idx = jax.lax.top_k(logits, TOPK)
    gates = jax.nn.softmax(top_vals, axis=1)
    flat_e = top_idx.reshape(-1)
    onehot = (flat_e[:, None] == jnp.arange(E)[None, :]).astype(jnp.int32)
    csum = jnp.cumsum(onehot, axis=0)
    counts = csum[-1]
    rank = jnp.take_along_axis(csum, flat_e[:, None], axis=1)[:, 0] - 1
    padded_counts = ((counts + BLK - 1) // BLK) * BLK
    cum_pad = jnp.cumsum(padded_counts)
    pad_offset = cum_pad - padded_counts
    pos = pad_offset[flat_e] + rank
    starts = jnp.arange(NB, dtype=jnp.int32) * BLK
    block_expert = jnp.minimum(
        jnp.sum(starts[:, None] >= cum_pad[None, :], axis=1), E - 1
    ).astype(jnp.int32)

    pos2 = pos.reshape(T, TOPK)
    sub = jnp.arange(8, dtype=jnp.int32)[None, :]
    pos_even_sub = (pos2[:, 0:1] * 8 + sub).reshape(-1)
    pos_odd_sub = (pos2[:, 1:2] * 8 + sub).reshape(-1)
    x_sub = x.reshape(T * 8, d // 8)
    x_sorted = _sc_dispatch(x_sub, pos_even_sub, pos_odd_sub, P).reshape(P, d)

    w_in_b = w_in.astype(jnp.bfloat16)
    w_out_b = w_out.astype(jnp.bfloat16)
    grid_spec = pltpu.PrefetchScalarGridSpec(
        num_scalar_prefetch=1,
        grid=(NB,),
        in_specs=[
            pl.BlockSpec((BLK, d), lambda b, be: (b, 0)),
            pl.BlockSpec((1, 2 * FF, d), lambda b, be: (be[b], 0, 0)),
            pl.BlockSpec((1, d, FF), lambda b, be: (be[b], 0, 0)),
        ],
        out_specs=pl.BlockSpec((BLK, d), lambda b, be: (b, 0)),
    )
    y = pl.pallas_call(
        _moe_body,
        grid_spec=grid_spec,
        out_shape=jax.ShapeDtypeStruct((P, d), jnp.bfloat16),
    )(block_expert, x_sorted, w_in_b, w_out_b)

    y0 = y[pos2[:, 0]].astype(jnp.float32)
    y1 = y[pos2[:, 1]].astype(jnp.float32)
    out = gates[:, 0:1] * y0 + gates[:, 1:2] * y1
    return out.reshape(bsz, length, d), logits


# speedup vs baseline: 1.4718x; 1.4718x over previous
import jax
import jax.numpy as jnp
from jax.experimental import pallas as pl
from jax.experimental.pallas import tpu as pltpu
from jax.experimental.pallas import tpu_sc as plsc

FF = 2048
E = 8
TOPK = 2
BLK = 512
BM_ROUTER = 1024
SC_W = 128


def _router_body(x_ref, wr_ref, logits_ref, eidx_ref, gates_ref, rank_ref,
                 counts_ref, carry_ref, tri_ref):
    bm = x_ref.shape[0]
    x = x_ref[...].astype(jnp.bfloat16)
    w = wr_ref[...].astype(jnp.bfloat16)
    logits = jax.lax.dot_general(
        x, w, (((1,), (1,)), ((), ())), preferred_element_type=jnp.float32)
    logits_ref[...] = logits

    @pl.when(pl.program_id(0) == 0)
    def _init():
        carry_ref[...] = jnp.zeros_like(carry_ref)
        r = jax.lax.broadcasted_iota(jnp.int32, (bm, bm), 0)
        c = jax.lax.broadcasted_iota(jnp.int32, (bm, bm), 1)
        tri_ref[...] = (c <= r).astype(jnp.bfloat16)

    iota8 = jax.lax.broadcasted_iota(jnp.int32, (bm, E), 1)
    neg_inf = jnp.float32(-jnp.inf)

    m1 = jnp.max(logits, axis=1, keepdims=True)
    a1 = jnp.min(jnp.where(logits == m1, iota8, E), axis=1, keepdims=True)
    oh1 = iota8 == a1
    l2 = jnp.where(oh1, neg_inf, logits)
    m2 = jnp.max(l2, axis=1, keepdims=True)
    a2 = jnp.min(jnp.where(l2 == m2, iota8, E), axis=1, keepdims=True)
    oh2 = iota8 == a2

    ed = jnp.exp(m2 - m1)
    g1 = 1.0 / (1.0 + ed)
    eidx_ref[...] = jnp.concatenate([a1, a2], axis=1)
    gates_ref[...] = jnp.concatenate([g1, 1.0 - g1], axis=1)

    tri = tri_ref[...]
    oh1b = oh1.astype(jnp.bfloat16)
    oh2b = oh2.astype(jnp.bfloat16)
    csum1 = jax.lax.dot_general(
        tri, oh1b, (((1,), (0,)), ((), ())), preferred_element_type=jnp.float32)
    csum2 = jax.lax.dot_general(
        tri, oh2b, (((1,), (0,)), ((), ())), preferred_element_type=jnp.float32)
    cnt1 = csum1[bm - 1:bm, :]
    cnt2 = csum2[bm - 1:bm, :]
    carry = carry_ref[...]
    sel1 = jnp.sum(jnp.where(oh1, csum1 + carry, 0.0), axis=1, keepdims=True)
    sel2 = jnp.sum(jnp.where(oh2, csum2 + carry + cnt1, 0.0), axis=1,
                   keepdims=True)
    rank_ref[...] = jnp.concatenate([sel1 - 1.0, sel2 - 1.0],
                                    axis=1).astype(jnp.int32)
    new_carry = carry + cnt1 + cnt2
    carry_ref[...] = new_carry
    counts_ref[...] = new_carry.astype(jnp.int32)


def _moe_body(be_ref, x_ref, win_ref, wout_ref, out_ref):
    x = x_ref[...].astype(jnp.bfloat16)
    win = win_ref[0]
    h = jax.lax.dot_general(
        x, win, (((1,), (1,)), ((), ())), preferred_element_type=jnp.float32)
    g = h[:, :FF]
    u = h[:, FF:]
    a = (g * jax.nn.sigmoid(g) * u).astype(jnp.bfloat16)
    wout = wout_ref[0]
    out_ref[...] = jax.lax.dot_general(
        a, wout, (((1,), (1,)), ((), ())),
        preferred_element_type=jnp.float32)


def _sc_dispatch(x_sub, pos_even, pos_odd, P):
    n_sub, dsub = x_sub.shape
    mesh = plsc.VectorSubcoreMesh(core_axis_name="core", subcore_axis_name="subcore")

    @pl.kernel(out_type=jax.ShapeDtypeStruct((P * 8, dsub), x_sub.dtype), mesh=mesh)
    def dispatch_kernel(x_hbm, ie_hbm, io_hbm, o_hbm):
        def body(x_vmem, ie_vmem, io_vmem):
            pltpu.sync_copy(x_vmem, o_hbm.at[ie_vmem.at[0]])
            pltpu.sync_copy(x_vmem, o_hbm.at[io_vmem.at[0]])

        pltpu.emit_pipeline(
            body,
            grid=(n_sub // SC_W,),
            in_specs=[
                pl.BlockSpec((SC_W, dsub), lambda i: (i, 0)),
                pl.BlockSpec((1, SC_W), lambda i: (0, i)),
                pl.BlockSpec((1, SC_W), lambda i: (0, i)),
            ],
            out_specs=[],
            core_axis_name=("core", "subcore"),
            dimension_semantics=(pltpu.PARALLEL,),
        )(x_hbm, ie_hbm, io_hbm)

    return dispatch_kernel(x_sub, pos_even.reshape(1, n_sub), pos_odd.reshape(1, n_sub))


def kernel(layer_input, w_router, w_in, w_out):
    bsz, length, d = layer_input.shape
    T = bsz * length
    S = T * TOPK
    P = S + E * BLK
    NB = P // BLK
    x = layer_input.reshape(T, d)

    logits, top_idx, gates, rank, counts = pl.pallas_call(
        _router_body,
        grid=(T // BM_ROUTER,),
        in_specs=[
            pl.BlockSpec((BM_ROUTER, d), lambda i: (i, 0)),
            pl.BlockSpec((E, d), lambda i: (0, 0)),
        ],
        out_specs=[
            pl.BlockSpec((BM_ROUTER, E), lambda i: (i, 0)),
            pl.BlockSpec((BM_ROUTER, TOPK), lambda i: (i, 0)),
            pl.BlockSpec((BM_ROUTER, TOPK), lambda i: (i, 0)),
            pl.BlockSpec((BM_ROUTER, TOPK), lambda i: (i, 0)),
            pl.BlockSpec((1, E), lambda i: (0, 0)),
        ],
        out_shape=[
            jax.ShapeDtypeStruct((T, E), jnp.float32),
            jax.ShapeDtypeStruct((T, TOPK), jnp.int32),
            jax.ShapeDtypeStruct((T, TOPK), jnp.float32),
            jax.ShapeDtypeStruct((T, TOPK), jnp.int32),
            jax.ShapeDtypeStruct((1, E), jnp.int32),
        ],
        scratch_shapes=[
            pltpu.VMEM((1, E), jnp.float32),
            pltpu.VMEM((BM_ROUTER, BM_ROUTER), jnp.bfloat16),
        ],
    )(x, w_router)

    padded_counts = ((counts[0] + BLK - 1) // BLK) * BLK
    cum_pad = jnp.cumsum(padded_counts)
    pad_offset = cum_pad - padded_counts
    starts = jnp.arange(NB, dtype=jnp.int32) * BLK
    block_expert = jnp.minimum(
        jnp.sum(starts[:, None] >= cum_pad[None, :], axis=1), E - 1
    ).astype(jnp.int32)

    pos2 = jnp.take(pad_offset, top_idx) + rank
    sub = jnp.arange(8, dtype=jnp.int32)[None, :]
    pos_even_sub = (pos2[:, 0:1] * 8 + sub).reshape(-1)
    pos_odd_sub = (pos2[:, 1:2] * 8 + sub).reshape(-1)
    x_sub = x.reshape(T * 8, d // 8)
    x_sorted = _sc_dispatch(x_sub, pos_even_sub, pos_odd_sub, P).reshape(P, d)

    w_in_b = w_in.astype(jnp.bfloat16)
    w_out_b = w_out.astype(jnp.bfloat16)
    grid_spec = pltpu.PrefetchScalarGridSpec(
        num_scalar_prefetch=1,
        grid=(NB,),
        in_specs=[
            pl.BlockSpec((BLK, d), lambda b, be: (b, 0)),
            pl.BlockSpec((1, 2 * FF, d), lambda b, be: (be[b], 0, 0)),
            pl.BlockSpec((1, d, FF), lambda b, be: (be[b], 0, 0)),
        ],
        out_specs=pl.BlockSpec((BLK, d), lambda b, be: (b, 0)),
    )
    y = pl.pallas_call(
        _moe_body,
        grid_spec=grid_spec,
        out_shape=jax.ShapeDtypeStruct((P, d), jnp.float32),
    )(block_expert, x_sorted, w_in_b, w_out_b)

    y0 = y[pos2[:, 0]]
    y1 = y[pos2[:, 1]]
    out = gates[:, 0:1] * y0 + gates[:, 1:2] * y1
    return out.reshape(bsz, length, d), logits


# speedup vs baseline: 1.7642x; 1.1987x over previous
import jax
import jax.numpy as jnp
from jax.experimental import pallas as pl
from jax.experimental.pallas import tpu as pltpu
from jax.experimental.pallas import tpu_sc as plsc

FF = 2048
E = 8
TOPK = 2
BLK = 512
BM_ROUTER = 256
SC_W = 16


def _router_body(x_ref, wr_ref, logits_ref, eidx_ref, gates_ref, rank_ref,
                 counts_ref, carry_ref, tri_ref):
    bm = x_ref.shape[0]
    x = x_ref[...].astype(jnp.bfloat16)
    w = wr_ref[...].astype(jnp.bfloat16)
    logits = jax.lax.dot_general(
        x, w, (((1,), (1,)), ((), ())), preferred_element_type=jnp.float32)
    logits_ref[...] = logits

    @pl.when(pl.program_id(0) == 0)
    def _init():
        carry_ref[...] = jnp.zeros_like(carry_ref)
        r = jax.lax.broadcasted_iota(jnp.int32, (bm, bm), 0)
        c = jax.lax.broadcasted_iota(jnp.int32, (bm, bm), 1)
        tri_ref[...] = (c <= r).astype(jnp.bfloat16)

    iota8 = jax.lax.broadcasted_iota(jnp.int32, (bm, E), 1)
    neg_inf = jnp.float32(-jnp.inf)

    m1 = jnp.max(logits, axis=1, keepdims=True)
    a1 = jnp.min(jnp.where(logits == m1, iota8, E), axis=1, keepdims=True)
    oh1 = iota8 == a1
    l2 = jnp.where(oh1, neg_inf, logits)
    m2 = jnp.max(l2, axis=1, keepdims=True)
    a2 = jnp.min(jnp.where(l2 == m2, iota8, E), axis=1, keepdims=True)
    oh2 = iota8 == a2

    ed = jnp.exp(m2 - m1)
    g1 = 1.0 / (1.0 + ed)
    eidx_ref[...] = jnp.concatenate([a1, a2], axis=1)
    gates_ref[...] = jnp.concatenate([g1, 1.0 - g1], axis=1)

    tri = tri_ref[...]
    oh1b = oh1.astype(jnp.bfloat16)
    oh2b = oh2.astype(jnp.bfloat16)
    csum1 = jax.lax.dot_general(
        tri, oh1b, (((1,), (0,)), ((), ())), preferred_element_type=jnp.float32)
    csum2 = jax.lax.dot_general(
        tri, oh2b, (((1,), (0,)), ((), ())), preferred_element_type=jnp.float32)
    cnt1 = csum1[bm - 1:bm, :]
    cnt2 = csum2[bm - 1:bm, :]
    carry = carry_ref[...]
    sel1 = jnp.sum(jnp.where(oh1, csum1 + carry, 0.0), axis=1, keepdims=True)
    sel2 = jnp.sum(jnp.where(oh2, csum2 + carry + cnt1, 0.0), axis=1,
                   keepdims=True)
    rank_ref[...] = jnp.concatenate([sel1 - 1.0, sel2 - 1.0],
                                    axis=1).astype(jnp.int32)
    new_carry = carry + cnt1 + cnt2
    carry_ref[...] = new_carry
    counts_ref[...] = new_carry.astype(jnp.int32)


def _moe_body(be_ref, x_ref, win_ref, wout_ref, out_ref):
    x = x_ref[...].astype(jnp.bfloat16)
    win = win_ref[0]
    h = jax.lax.dot_general(
        x, win, (((1,), (1,)), ((), ())), preferred_element_type=jnp.float32)
    g = h[:, :FF]
    u = h[:, FF:]
    a = (g * jax.nn.sigmoid(g) * u).astype(jnp.bfloat16)
    wout = wout_ref[0]
    out_ref[...] = jax.lax.dot_general(
        a, wout, (((1,), (1,)), ((), ())),
        preferred_element_type=jnp.float32)


def _sc_dispatch(x, idx_even, idx_odd, P):
    T, d = x.shape
    nw = T // SC_W
    mesh = plsc.VectorSubcoreMesh(core_axis_name="core", subcore_axis_name="subcore")

    @pl.kernel(out_type=jax.ShapeDtypeStruct((P, d), x.dtype), mesh=mesh)
    def dispatch_kernel(x_hbm, ie_hbm, io_hbm, o_hbm):
        def body(x_vmem, ie_vmem, io_vmem):
            pltpu.sync_copy(x_vmem, o_hbm.at[ie_vmem.at[0, 0, pl.ds(0, SC_W)]])
            pltpu.sync_copy(x_vmem, o_hbm.at[io_vmem.at[0, 0, pl.ds(0, SC_W)]])

        pltpu.emit_pipeline(
            body,
            grid=(nw,),
            in_specs=[
                pl.BlockSpec((SC_W, d), lambda i: (i, 0)),
                pl.BlockSpec((1, 1, 128), lambda i: (i, 0, 0)),
                pl.BlockSpec((1, 1, 128), lambda i: (i, 0, 0)),
            ],
            out_specs=[],
            core_axis_name=("core", "subcore"),
            dimension_semantics=(pltpu.PARALLEL,),
        )(x_hbm, ie_hbm, io_hbm)

    return dispatch_kernel(x, idx_even, idx_odd)


def _cast_body(win_ref, wout_ref, win_b_ref, wout_b_ref):
    win_b_ref[...] = win_ref[...].astype(jnp.bfloat16)
    wout_b_ref[...] = wout_ref[...].astype(jnp.bfloat16)


def kernel(layer_input, w_router, w_in, w_out):
    bsz, length, d = layer_input.shape
    T = bsz * length
    S = T * TOPK
    P = S + E * BLK
    NB = P // BLK
    x = layer_input.reshape(T, d)

    logits, top_idx, gates, rank, counts = pl.pallas_call(
        _router_body,
        grid=(T // BM_ROUTER,),
        in_specs=[
            pl.BlockSpec((BM_ROUTER, d), lambda i: (i, 0)),
            pl.BlockSpec((E, d), lambda i: (0, 0)),
        ],
        out_specs=[
            pl.BlockSpec((BM_ROUTER, E), lambda i: (i, 0)),
            pl.BlockSpec((BM_ROUTER, TOPK), lambda i: (i, 0)),
            pl.BlockSpec((BM_ROUTER, TOPK), lambda i: (i, 0)),
            pl.BlockSpec((BM_ROUTER, TOPK), lambda i: (i, 0)),
            pl.BlockSpec((1, E), lambda i: (0, 0)),
        ],
        out_shape=[
            jax.ShapeDtypeStruct((T, E), jnp.float32),
            jax.ShapeDtypeStruct((T, TOPK), jnp.int32),
            jax.ShapeDtypeStruct((T, TOPK), jnp.float32),
            jax.ShapeDtypeStruct((T, TOPK), jnp.int32),
            jax.ShapeDtypeStruct((1, E), jnp.int32),
        ],
        scratch_shapes=[
            pltpu.VMEM((1, E), jnp.float32),
            pltpu.VMEM((BM_ROUTER, BM_ROUTER), jnp.bfloat16),
        ],
    )(x, w_router)

    padded_counts = ((counts[0] + BLK - 1) // BLK) * BLK
    cum_pad = jnp.cumsum(padded_counts)
    pad_offset = cum_pad - padded_counts
    starts = jnp.arange(NB, dtype=jnp.int32) * BLK
    block_expert = jnp.minimum(
        jnp.sum(starts[:, None] >= cum_pad[None, :], axis=1), E - 1
    ).astype(jnp.int32)

    pos2 = jnp.take(pad_offset, top_idx) + rank
    nw = T // SC_W
    idx_even = jnp.pad(pos2[:, 0].reshape(nw, 1, SC_W), ((0, 0), (0, 0), (0, 128 - SC_W)))
    idx_odd = jnp.pad(pos2[:, 1].reshape(nw, 1, SC_W), ((0, 0), (0, 0), (0, 128 - SC_W)))
    x_sorted = _sc_dispatch(x, idx_even, idx_odd, P)

    CH = 16
    win_f = w_in.reshape(CH, E * 2 * FF * d // CH // 1024, 1024)
    wout_f = w_out.reshape(CH, E * d * FF // CH // 2048, 2048)
    w_in_b, w_out_b = pl.pallas_call(
        _cast_body,
        grid=(CH,),
        in_specs=[
            pl.BlockSpec((1,) + win_f.shape[1:], lambda i: (i, 0, 0)),
            pl.BlockSpec((1,) + wout_f.shape[1:], lambda i: (i, 0, 0)),
        ],
        out_specs=[
            pl.BlockSpec((1,) + win_f.shape[1:], lambda i: (i, 0, 0)),
            pl.BlockSpec((1,) + wout_f.shape[1:], lambda i: (i, 0, 0)),
        ],
        out_shape=[
            jax.ShapeDtypeStruct(win_f.shape, jnp.bfloat16),
            jax.ShapeDtypeStruct(wout_f.shape, jnp.bfloat16),
        ],
    )(win_f, wout_f)
    w_in_b = w_in_b.reshape(E, 2 * FF, d)
    w_out_b = w_out_b.reshape(E, d, FF)
    grid_spec = pltpu.PrefetchScalarGridSpec(
        num_scalar_prefetch=1,
        grid=(NB,),
        in_specs=[
            pl.BlockSpec((BLK, d), lambda b, be: (b, 0)),
            pl.BlockSpec((1, 2 * FF, d), lambda b, be: (be[b], 0, 0)),
            pl.BlockSpec((1, d, FF), lambda b, be: (be[b], 0, 0)),
        ],
        out_specs=pl.BlockSpec((BLK, d), lambda b, be: (b, 0)),
    )
    y = pl.pallas_call(
        _moe_body,
        grid_spec=grid_spec,
        out_shape=jax.ShapeDtypeStruct((P, d), jnp.float32),
    )(block_expert, x_sorted, w_in_b, w_out_b)

    y0 = y[pos2[:, 0]]
    y1 = y[pos2[:, 1]]
    out = gates[:, 0:1] * y0 + gates[:, 1:2] * y1
    return out.reshape(bsz, length, d), logits


# speedup vs baseline: 1.8165x; 1.0296x over previous
import jax
import jax.numpy as jnp
from jax.experimental import pallas as pl
from jax.experimental.pallas import tpu as pltpu
from jax.experimental.pallas import tpu_sc as plsc

FF = 2048
E = 8
TOPK = 2
BLK = 256
BM_ROUTER = 512
SC_W = 16


def _router_body(x_ref, wr_ref, logits_ref, eidx_ref, gates_ref, rank_ref,
                 counts_ref, carry_ref, tri_ref):
    bm = x_ref.shape[0]
    x = x_ref[...].astype(jnp.bfloat16)
    w = wr_ref[...].astype(jnp.bfloat16)
    logits = jax.lax.dot_general(
        x, w, (((1,), (1,)), ((), ())), preferred_element_type=jnp.float32)
    logits_ref[...] = logits

    @pl.when(pl.program_id(0) == 0)
    def _init():
        carry_ref[...] = jnp.zeros_like(carry_ref)
        r = jax.lax.broadcasted_iota(jnp.int32, (bm, bm), 0)
        c = jax.lax.broadcasted_iota(jnp.int32, (bm, bm), 1)
        tri_ref[...] = (c <= r).astype(jnp.bfloat16)

    iota8 = jax.lax.broadcasted_iota(jnp.int32, (bm, E), 1)
    neg_inf = jnp.float32(-jnp.inf)

    m1 = jnp.max(logits, axis=1, keepdims=True)
    a1 = jnp.min(jnp.where(logits == m1, iota8, E), axis=1, keepdims=True)
    oh1 = iota8 == a1
    l2 = jnp.where(oh1, neg_inf, logits)
    m2 = jnp.max(l2, axis=1, keepdims=True)
    a2 = jnp.min(jnp.where(l2 == m2, iota8, E), axis=1, keepdims=True)
    oh2 = iota8 == a2

    ed = jnp.exp(m2 - m1)
    g1 = 1.0 / (1.0 + ed)
    eidx_ref[...] = jnp.concatenate([a1, a2], axis=1)
    gates_ref[...] = jnp.concatenate([g1, 1.0 - g1], axis=1)

    tri = tri_ref[...]
    oh1b = oh1.astype(jnp.bfloat16)
    oh2b = oh2.astype(jnp.bfloat16)
    csum1 = jax.lax.dot_general(
        tri, oh1b, (((1,), (0,)), ((), ())), preferred_element_type=jnp.float32)
    csum2 = jax.lax.dot_general(
        tri, oh2b, (((1,), (0,)), ((), ())), preferred_element_type=jnp.float32)
    cnt1 = csum1[bm - 1:bm, :]
    cnt2 = csum2[bm - 1:bm, :]
    carry = carry_ref[...]
    sel1 = jnp.sum(jnp.where(oh1, csum1 + carry, 0.0), axis=1, keepdims=True)
    sel2 = jnp.sum(jnp.where(oh2, csum2 + carry + cnt1, 0.0), axis=1,
                   keepdims=True)
    rank_ref[...] = jnp.concatenate([sel1 - 1.0, sel2 - 1.0],
                                    axis=1).astype(jnp.int32)
    new_carry = carry + cnt1 + cnt2
    carry_ref[...] = new_carry
    counts_ref[...] = new_carry.astype(jnp.int32)


def _moe_body(be_ref, x_ref, win_ref, wout_ref, out_ref):
    x = x_ref[...].astype(jnp.bfloat16)
    win = win_ref[0]
    h = jax.lax.dot_general(
        x, win, (((1,), (1,)), ((), ())), preferred_element_type=jnp.float32)
    g = h[:, :FF]
    u = h[:, FF:]
    a = (g * jax.nn.sigmoid(g) * u).astype(jnp.bfloat16)
    wout = wout_ref[0]
    out_ref[...] = jax.lax.dot_general(
        a, wout, (((1,), (1,)), ((), ())),
        preferred_element_type=jnp.float32)


def _sc_dispatch(x, idx_even, idx_odd, P):
    T, d = x.shape
    nw = T // SC_W
    mesh = plsc.VectorSubcoreMesh(core_axis_name="core", subcore_axis_name="subcore")

    @pl.kernel(out_type=jax.ShapeDtypeStruct((P, d), x.dtype), mesh=mesh)
    def dispatch_kernel(x_hbm, ie_hbm, io_hbm, o_hbm):
        def body(x_vmem, ie_vmem, io_vmem):
            pltpu.sync_copy(x_vmem, o_hbm.at[ie_vmem.at[0, 0, pl.ds(0, SC_W)]])
            pltpu.sync_copy(x_vmem, o_hbm.at[io_vmem.at[0, 0, pl.ds(0, SC_W)]])

        pltpu.emit_pipeline(
            body,
            grid=(nw,),
            in_specs=[
                pl.BlockSpec((SC_W, d), lambda i: (i, 0)),
                pl.BlockSpec((1, 1, 128), lambda i: (i, 0, 0)),
                pl.BlockSpec((1, 1, 128), lambda i: (i, 0, 0)),
            ],
            out_specs=[],
            core_axis_name=("core", "subcore"),
            dimension_semantics=(pltpu.PARALLEL,),
        )(x_hbm, ie_hbm, io_hbm)

    return dispatch_kernel(x, idx_even, idx_odd)


def _cast_body(win_ref, wout_ref, win_b_ref, wout_b_ref):
    win_b_ref[...] = win_ref[...].astype(jnp.bfloat16)
    wout_b_ref[...] = wout_ref[...].astype(jnp.bfloat16)


def kernel(layer_input, w_router, w_in, w_out):
    bsz, length, d = layer_input.shape
    T = bsz * length
    S = T * TOPK
    P = S + E * BLK
    NB = P // BLK
    x = layer_input.reshape(T, d)

    logits, top_idx, gates, rank, counts = pl.pallas_call(
        _router_body,
        grid=(T // BM_ROUTER,),
        in_specs=[
            pl.BlockSpec((BM_ROUTER, d), lambda i: (i, 0)),
            pl.BlockSpec((E, d), lambda i: (0, 0)),
        ],
        out_specs=[
            pl.BlockSpec((BM_ROUTER, E), lambda i: (i, 0)),
            pl.BlockSpec((BM_ROUTER, TOPK), lambda i: (i, 0)),
            pl.BlockSpec((BM_ROUTER, TOPK), lambda i: (i, 0)),
            pl.BlockSpec((BM_ROUTER, TOPK), lambda i: (i, 0)),
            pl.BlockSpec((1, E), lambda i: (0, 0)),
        ],
        out_shape=[
            jax.ShapeDtypeStruct((T, E), jnp.float32),
            jax.ShapeDtypeStruct((T, TOPK), jnp.int32),
            jax.ShapeDtypeStruct((T, TOPK), jnp.float32),
            jax.ShapeDtypeStruct((T, TOPK), jnp.int32),
            jax.ShapeDtypeStruct((1, E), jnp.int32),
        ],
        scratch_shapes=[
            pltpu.VMEM((1, E), jnp.float32),
            pltpu.VMEM((BM_ROUTER, BM_ROUTER), jnp.bfloat16),
        ],
    )(x, w_router)

    padded_counts = ((counts[0] + BLK - 1) // BLK) * BLK
    cum_pad = jnp.cumsum(padded_counts)
    pad_offset = cum_pad - padded_counts
    starts = jnp.arange(NB, dtype=jnp.int32) * BLK
    block_expert = jnp.minimum(
        jnp.sum(starts[:, None] >= cum_pad[None, :], axis=1), E - 1
    ).astype(jnp.int32)

    pos2 = jnp.take(pad_offset, top_idx) + rank
    nw = T // SC_W
    idx_even = jnp.pad(pos2[:, 0].reshape(nw, 1, SC_W), ((0, 0), (0, 0), (0, 128 - SC_W)))
    idx_odd = jnp.pad(pos2[:, 1].reshape(nw, 1, SC_W), ((0, 0), (0, 0), (0, 128 - SC_W)))
    x_sorted = _sc_dispatch(x, idx_even, idx_odd, P)

    CH = 16
    win_f = w_in.reshape(CH, E * 2 * FF * d // CH // 1024, 1024)
    wout_f = w_out.reshape(CH, E * d * FF // CH // 2048, 2048)
    w_in_b, w_out_b = pl.pallas_call(
        _cast_body,
        grid=(CH,),
        in_specs=[
            pl.BlockSpec((1,) + win_f.shape[1:], lambda i: (i, 0, 0)),
            pl.BlockSpec((1,) + wout_f.shape[1:], lambda i: (i, 0, 0)),
        ],
        out_specs=[
            pl.BlockSpec((1,) + win_f.shape[1:], lambda i: (i, 0, 0)),
            pl.BlockSpec((1,) + wout_f.shape[1:], lambda i: (i, 0, 0)),
        ],
        out_shape=[
            jax.ShapeDtypeStruct(win_f.shape, jnp.bfloat16),
            jax.ShapeDtypeStruct(wout_f.shape, jnp.bfloat16),
        ],
    )(win_f, wout_f)
    w_in_b = w_in_b.reshape(E, 2 * FF, d)
    w_out_b = w_out_b.reshape(E, d, FF)
    grid_spec = pltpu.PrefetchScalarGridSpec(
        num_scalar_prefetch=1,
        grid=(NB,),
        in_specs=[
            pl.BlockSpec((BLK, d), lambda b, be: (b, 0)),
            pl.BlockSpec((1, 2 * FF, d), lambda b, be: (be[b], 0, 0)),
            pl.BlockSpec((1, d, FF), lambda b, be: (be[b], 0, 0)),
        ],
        out_specs=pl.BlockSpec((BLK, d), lambda b, be: (b, 0)),
    )
    y = pl.pallas_call(
        _moe_body,
        grid_spec=grid_spec,
        out_shape=jax.ShapeDtypeStruct((P, d), jnp.float32),
    )(block_expert, x_sorted, w_in_b, w_out_b)

    y0 = y[pos2[:, 0]]
    y1 = y[pos2[:, 1]]
    out = gates[:, 0:1] * y0 + gates[:, 1:2] * y1
    return out.reshape(bsz, length, d), logits


# speedup vs baseline: 1.8830x; 1.0366x over previous
import jax
import jax.numpy as jnp
from jax.experimental import pallas as pl
from jax.experimental.pallas import tpu as pltpu
from jax.experimental.pallas import tpu_sc as plsc

FF = 2048
E = 8
TOPK = 2
BLK = 256
BM_ROUTER = 512
SC_W = 16


def _router_body(x_ref, wr_ref, logits_ref, eidx_ref, gates_ref, rank_ref,
                 counts_ref, carry_ref, tri_ref):
    bm = x_ref.shape[0]
    x = x_ref[...].astype(jnp.bfloat16)
    w = wr_ref[...].astype(jnp.bfloat16)
    logits = jax.lax.dot_general(
        x, w, (((1,), (1,)), ((), ())), preferred_element_type=jnp.float32)
    logits_ref[...] = logits

    @pl.when(pl.program_id(0) == 0)
    def _init():
        carry_ref[...] = jnp.zeros_like(carry_ref)
        r = jax.lax.broadcasted_iota(jnp.int32, (bm, bm), 0)
        c = jax.lax.broadcasted_iota(jnp.int32, (bm, bm), 1)
        tri_ref[...] = (c <= r).astype(jnp.bfloat16)

    iota8 = jax.lax.broadcasted_iota(jnp.int32, (bm, E), 1)
    neg_inf = jnp.float32(-jnp.inf)

    m1 = jnp.max(logits, axis=1, keepdims=True)
    a1 = jnp.min(jnp.where(logits == m1, iota8, E), axis=1, keepdims=True)
    oh1 = iota8 == a1
    l2 = jnp.where(oh1, neg_inf, logits)
    m2 = jnp.max(l2, axis=1, keepdims=True)
    a2 = jnp.min(jnp.where(l2 == m2, iota8, E), axis=1, keepdims=True)
    oh2 = iota8 == a2

    ed = jnp.exp(m2 - m1)
    g1 = 1.0 / (1.0 + ed)
    eidx_ref[...] = jnp.concatenate([a1, a2], axis=1)
    gates_ref[...] = jnp.concatenate([g1, 1.0 - g1], axis=1)

    tri = tri_ref[...]
    oh1b = oh1.astype(jnp.bfloat16)
    oh2b = oh2.astype(jnp.bfloat16)
    csum1 = jax.lax.dot_general(
        tri, oh1b, (((1,), (0,)), ((), ())), preferred_element_type=jnp.float32)
    csum2 = jax.lax.dot_general(
        tri, oh2b, (((1,), (0,)), ((), ())), preferred_element_type=jnp.float32)
    cnt1 = csum1[bm - 1:bm, :]
    cnt2 = csum2[bm - 1:bm, :]
    carry = carry_ref[...]
    sel1 = jnp.sum(jnp.where(oh1, csum1 + carry, 0.0), axis=1, keepdims=True)
    sel2 = jnp.sum(jnp.where(oh2, csum2 + carry + cnt1, 0.0), axis=1,
                   keepdims=True)
    rank_ref[...] = jnp.concatenate([sel1 - 1.0, sel2 - 1.0],
                                    axis=1).astype(jnp.int32)
    new_carry = carry + cnt1 + cnt2
    carry_ref[...] = new_carry
    counts_ref[...] = new_carry.astype(jnp.int32)


def _moe_body(be_ref, x_ref, win_ref, wout_ref, out_ref):
    x = x_ref[...].astype(jnp.bfloat16)
    win = win_ref[0]
    h = jax.lax.dot_general(
        x, win, (((1,), (1,)), ((), ())), preferred_element_type=jnp.float32)
    g = h[:, :FF]
    u = h[:, FF:]
    a = (g * jax.nn.sigmoid(g) * u).astype(jnp.bfloat16)
    wout = wout_ref[0].astype(jnp.bfloat16)
    out_ref[...] = jax.lax.dot_general(
        a, wout, (((1,), (1,)), ((), ())),
        preferred_element_type=jnp.float32)


def _sc_dispatch(x, idx_even, idx_odd, P):
    T, d = x.shape
    nw = T // SC_W
    mesh = plsc.VectorSubcoreMesh(core_axis_name="core", subcore_axis_name="subcore")

    @pl.kernel(out_type=jax.ShapeDtypeStruct((P, d), x.dtype), mesh=mesh)
    def dispatch_kernel(x_hbm, ie_hbm, io_hbm, o_hbm):
        def body(x_vmem, ie_vmem, io_vmem):
            pltpu.sync_copy(x_vmem, o_hbm.at[ie_vmem.at[0, 0, pl.ds(0, SC_W)]])
            pltpu.sync_copy(x_vmem, o_hbm.at[io_vmem.at[0, 0, pl.ds(0, SC_W)]])

        pltpu.emit_pipeline(
            body,
            grid=(nw,),
            in_specs=[
                pl.BlockSpec((SC_W, d), lambda i: (i, 0)),
                pl.BlockSpec((1, 1, 128), lambda i: (i, 0, 0)),
                pl.BlockSpec((1, 1, 128), lambda i: (i, 0, 0)),
            ],
            out_specs=[],
            core_axis_name=("core", "subcore"),
            dimension_semantics=(pltpu.PARALLEL,),
        )(x_hbm, ie_hbm, io_hbm)

    return dispatch_kernel(x, idx_even, idx_odd)


def _cast_body(win_ref, win_b_ref):
    win_b_ref[...] = win_ref[...].astype(jnp.bfloat16)


def kernel(layer_input, w_router, w_in, w_out):
    bsz, length, d = layer_input.shape
    T = bsz * length
    S = T * TOPK
    P = S + E * BLK
    NB = P // BLK
    x = layer_input.reshape(T, d)

    logits, top_idx, gates, rank, counts = pl.pallas_call(
        _router_body,
        grid=(T // BM_ROUTER,),
        in_specs=[
            pl.BlockSpec((BM_ROUTER, d), lambda i: (i, 0)),
            pl.BlockSpec((E, d), lambda i: (0, 0)),
        ],
        out_specs=[
            pl.BlockSpec((BM_ROUTER, E), lambda i: (i, 0)),
            pl.BlockSpec((BM_ROUTER, TOPK), lambda i: (i, 0)),
            pl.BlockSpec((BM_ROUTER, TOPK), lambda i: (i, 0)),
            pl.BlockSpec((BM_ROUTER, TOPK), lambda i: (i, 0)),
            pl.BlockSpec((1, E), lambda i: (0, 0)),
        ],
        out_shape=[
            jax.ShapeDtypeStruct((T, E), jnp.float32),
            jax.ShapeDtypeStruct((T, TOPK), jnp.int32),
            jax.ShapeDtypeStruct((T, TOPK), jnp.float32),
            jax.ShapeDtypeStruct((T, TOPK), jnp.int32),
            jax.ShapeDtypeStruct((1, E), jnp.int32),
        ],
        scratch_shapes=[
            pltpu.VMEM((1, E), jnp.float32),
            pltpu.VMEM((BM_ROUTER, BM_ROUTER), jnp.bfloat16),
        ],
    )(x, w_router)

    padded_counts = ((counts[0] + BLK - 1) // BLK) * BLK
    cum_pad = jnp.cumsum(padded_counts)
    pad_offset = cum_pad - padded_counts
    starts = jnp.arange(NB, dtype=jnp.int32) * BLK
    block_expert = jnp.minimum(
        jnp.sum(starts[:, None] >= cum_pad[None, :], axis=1), E - 1
    ).astype(jnp.int32)

    pos2 = jnp.take(pad_offset, top_idx) + rank
    nw = T // SC_W
    idx_even = jnp.pad(pos2[:, 0].reshape(nw, 1, SC_W), ((0, 0), (0, 0), (0, 128 - SC_W)))
    idx_odd = jnp.pad(pos2[:, 1].reshape(nw, 1, SC_W), ((0, 0), (0, 0), (0, 128 - SC_W)))
    x_sorted = _sc_dispatch(x, idx_even, idx_odd, P)

    CH = 16
    win_f = w_in.reshape(CH, E * 2 * FF * d // CH // 1024, 1024)
    w_in_b = pl.pallas_call(
        _cast_body,
        grid=(CH,),
        in_specs=[pl.BlockSpec((1,) + win_f.shape[1:], lambda i: (i, 0, 0))],
        out_specs=pl.BlockSpec((1,) + win_f.shape[1:], lambda i: (i, 0, 0)),
        out_shape=jax.ShapeDtypeStruct(win_f.shape, jnp.bfloat16),
    )(win_f)
    w_in_b = w_in_b.reshape(E, 2 * FF, d)
    w_out_b = w_out
    grid_spec = pltpu.PrefetchScalarGridSpec(
        num_scalar_prefetch=1,
        grid=(NB,),
        in_specs=[
            pl.BlockSpec((BLK, d), lambda b, be: (b, 0)),
            pl.BlockSpec((1, 2 * FF, d), lambda b, be: (be[b], 0, 0)),
            pl.BlockSpec((1, d, FF), lambda b, be: (be[b], 0, 0)),
        ],
        out_specs=pl.BlockSpec((BLK, d), lambda b, be: (b, 0)),
    )
    y = pl.pallas_call(
        _moe_body,
        grid_spec=grid_spec,
        out_shape=jax.ShapeDtypeStruct((P, d), jnp.float32),
    )(block_expert, x_sorted, w_in_b, w_out_b)

    y0 = y[pos2[:, 0]]
    y1 = y[pos2[:, 1]]
    out = gates[:, 0:1] * y0 + gates[:, 1:2] * y1
    return out.reshape(bsz, length, d), logits


# speedup vs baseline: 1.8838x; 1.0004x over previous
import jax
import jax.numpy as jnp
from jax.experimental import pallas as pl
from jax.experimental.pallas import tpu as pltpu
from jax.experimental.pallas import tpu_sc as plsc

FF = 2048
E = 8
TOPK = 2
BLK = 512
BM_ROUTER = 512
SC_W = 16


def _router_body(x_ref, wr_ref, logits_ref, eidx_ref, gates_ref, rank_ref,
                 counts_ref, carry_ref, tri_ref):
    bm = x_ref.shape[0]
    x = x_ref[...].astype(jnp.bfloat16)
    w = wr_ref[...].astype(jnp.bfloat16)
    logits = jax.lax.dot_general(
        x, w, (((1,), (1,)), ((), ())), preferred_element_type=jnp.float32)
    logits_ref[...] = logits

    @pl.when(pl.program_id(0) == 0)
    def _init():
        carry_ref[...] = jnp.zeros_like(carry_ref)
        r = jax.lax.broadcasted_iota(jnp.int32, (bm, bm), 0)
        c = jax.lax.broadcasted_iota(jnp.int32, (bm, bm), 1)
        tri_ref[...] = (c <= r).astype(jnp.bfloat16)

    iota8 = jax.lax.broadcasted_iota(jnp.int32, (bm, E), 1)
    neg_inf = jnp.float32(-jnp.inf)

    m1 = jnp.max(logits, axis=1, keepdims=True)
    a1 = jnp.min(jnp.where(logits == m1, iota8, E), axis=1, keepdims=True)
    oh1 = iota8 == a1
    l2 = jnp.where(oh1, neg_inf, logits)
    m2 = jnp.max(l2, axis=1, keepdims=True)
    a2 = jnp.min(jnp.where(l2 == m2, iota8, E), axis=1, keepdims=True)
    oh2 = iota8 == a2

    ed = jnp.exp(m2 - m1)
    g1 = 1.0 / (1.0 + ed)
    eidx_ref[...] = jnp.concatenate([a1, a2], axis=1)
    gates_ref[...] = jnp.concatenate([g1, 1.0 - g1], axis=1)

    tri = tri_ref[...]
    oh1b = oh1.astype(jnp.bfloat16)
    oh2b = oh2.astype(jnp.bfloat16)
    csum1 = jax.lax.dot_general(
        tri, oh1b, (((1,), (0,)), ((), ())), preferred_element_type=jnp.float32)
    csum2 = jax.lax.dot_general(
        tri, oh2b, (((1,), (0,)), ((), ())), preferred_element_type=jnp.float32)
    cnt1 = csum1[bm - 1:bm, :]
    cnt2 = csum2[bm - 1:bm, :]
    carry = carry_ref[...]
    sel1 = jnp.sum(jnp.where(oh1, csum1 + carry, 0.0), axis=1, keepdims=True)
    sel2 = jnp.sum(jnp.where(oh2, csum2 + carry + cnt1, 0.0), axis=1,
                   keepdims=True)
    rank_ref[...] = jnp.concatenate([sel1 - 1.0, sel2 - 1.0],
                                    axis=1).astype(jnp.int32)
    new_carry = carry + cnt1 + cnt2
    carry_ref[...] = new_carry
    counts_ref[...] = new_carry.astype(jnp.int32)


def _moe_body(be_ref, x_ref, win_ref, wout_ref, out_ref):
    x = x_ref[...].astype(jnp.bfloat16)
    win = win_ref[0]
    h = jax.lax.dot_general(
        x, win, (((1,), (1,)), ((), ())), preferred_element_type=jnp.float32)
    g = h[:, :FF]
    u = h[:, FF:]
    a = (g * jax.nn.sigmoid(g) * u).astype(jnp.bfloat16)
    wout = wout_ref[0].astype(jnp.bfloat16)
    out_ref[...] = jax.lax.dot_general(
        a, wout, (((1,), (1,)), ((), ())),
        preferred_element_type=jnp.float32)


def _sc_dispatch(x, idx_even, idx_odd, P):
    T, d = x.shape
    nw = T // SC_W
    mesh = plsc.VectorSubcoreMesh(core_axis_name="core", subcore_axis_name="subcore")

    @pl.kernel(out_type=jax.ShapeDtypeStruct((P, d), x.dtype), mesh=mesh)
    def dispatch_kernel(x_hbm, ie_hbm, io_hbm, o_hbm):
        def body(x_vmem, ie_vmem, io_vmem):
            pltpu.sync_copy(x_vmem, o_hbm.at[ie_vmem.at[0, 0, pl.ds(0, SC_W)]])
            pltpu.sync_copy(x_vmem, o_hbm.at[io_vmem.at[0, 0, pl.ds(0, SC_W)]])

        pltpu.emit_pipeline(
            body,
            grid=(nw,),
            in_specs=[
                pl.BlockSpec((SC_W, d), lambda i: (i, 0)),
                pl.BlockSpec((1, 1, 128), lambda i: (i, 0, 0)),
                pl.BlockSpec((1, 1, 128), lambda i: (i, 0, 0)),
            ],
            out_specs=[],
            core_axis_name=("core", "subcore"),
            dimension_semantics=(pltpu.PARALLEL,),
        )(x_hbm, ie_hbm, io_hbm)

    return dispatch_kernel(x, idx_even, idx_odd)


def _cast_body(win_ref, win_b_ref):
    win_b_ref[...] = win_ref[...].astype(jnp.bfloat16)


def kernel(layer_input, w_router, w_in, w_out):
    bsz, length, d = layer_input.shape
    T = bsz * length
    S = T * TOPK
    P = S + E * BLK
    NB = P // BLK
    x = layer_input.reshape(T, d)

    logits, top_idx, gates, rank, counts = pl.pallas_call(
        _router_body,
        grid=(T // BM_ROUTER,),
        in_specs=[
            pl.BlockSpec((BM_ROUTER, d), lambda i: (i, 0)),
            pl.BlockSpec((E, d), lambda i: (0, 0)),
        ],
        out_specs=[
            pl.BlockSpec((BM_ROUTER, E), lambda i: (i, 0)),
            pl.BlockSpec((BM_ROUTER, TOPK), lambda i: (i, 0)),
            pl.BlockSpec((BM_ROUTER, TOPK), lambda i: (i, 0)),
            pl.BlockSpec((BM_ROUTER, TOPK), lambda i: (i, 0)),
            pl.BlockSpec((1, E), lambda i: (0, 0)),
        ],
        out_shape=[
            jax.ShapeDtypeStruct((T, E), jnp.float32),
            jax.ShapeDtypeStruct((T, TOPK), jnp.int32),
            jax.ShapeDtypeStruct((T, TOPK), jnp.float32),
            jax.ShapeDtypeStruct((T, TOPK), jnp.int32),
            jax.ShapeDtypeStruct((1, E), jnp.int32),
        ],
        scratch_shapes=[
            pltpu.VMEM((1, E), jnp.float32),
            pltpu.VMEM((BM_ROUTER, BM_ROUTER), jnp.bfloat16),
        ],
    )(x, w_router)

    padded_counts = ((counts[0] + BLK - 1) // BLK) * BLK
    cum_pad = jnp.cumsum(padded_counts)
    pad_offset = cum_pad - padded_counts
    starts = jnp.arange(NB, dtype=jnp.int32) * BLK
    block_expert = jnp.minimum(
        jnp.sum(starts[:, None] >= cum_pad[None, :], axis=1), E - 1
    ).astype(jnp.int32)

    pos2 = jnp.take(pad_offset, top_idx) + rank
    nw = T // SC_W
    idx_even = jnp.pad(pos2[:, 0].reshape(nw, 1, SC_W), ((0, 0), (0, 0), (0, 128 - SC_W)))
    idx_odd = jnp.pad(pos2[:, 1].reshape(nw, 1, SC_W), ((0, 0), (0, 0), (0, 128 - SC_W)))
    x_sorted = _sc_dispatch(x, idx_even, idx_odd, P)

    CH = 16
    win_f = w_in.reshape(CH, E * 2 * FF * d // CH // 1024, 1024)
    w_in_b = pl.pallas_call(
        _cast_body,
        grid=(CH,),
        in_specs=[pl.BlockSpec((1,) + win_f.shape[1:], lambda i: (i, 0, 0))],
        out_specs=pl.BlockSpec((1,) + win_f.shape[1:], lambda i: (i, 0, 0)),
        out_shape=jax.ShapeDtypeStruct(win_f.shape, jnp.bfloat16),
    )(win_f)
    w_in_b = w_in_b.reshape(E, 2 * FF, d)
    w_out_b = w_out
    grid_spec = pltpu.PrefetchScalarGridSpec(
        num_scalar_prefetch=1,
        grid=(NB,),
        in_specs=[
            pl.BlockSpec((BLK, d), lambda b, be: (b, 0)),
            pl.BlockSpec((1, 2 * FF, d), lambda b, be: (be[b], 0, 0)),
            pl.BlockSpec((1, d, FF), lambda b, be: (be[b], 0, 0)),
        ],
        out_specs=pl.BlockSpec((BLK, d), lambda b, be: (b, 0)),
    )
    y = pl.pallas_call(
        _moe_body,
        grid_spec=grid_spec,
        out_shape=jax.ShapeDtypeStruct((P, d), jnp.float32),
    )(block_expert, x_sorted, w_in_b, w_out_b)

    y0 = y[pos2[:, 0]]
    y1 = y[pos2[:, 1]]
    out = gates[:, 0:1] * y0 + gates[:, 1:2] * y1
    return out.reshape(bsz, length, d), logits
